# baseline (speedup 1.0000x reference)
import numpy as np
import jax
import jax.numpy as jnp
from jax.experimental import pallas as pl
from jax.experimental.pallas import tpu as pltpu

_ANCHORS = np.array(
    [[10, 13], [16, 30], [33, 23], [30, 61], [62, 45], [59, 119],
     [116, 90], [156, 198], [373, 326]], dtype=np.float32)
_NUM_CLASSES = 80
_ATTRS = 5 + _NUM_CLASSES
_H = 76
_W = 76
_HW = _H * _W
_STRIDE = 608.0 / 76.0
_NBOX = 24
_NA = 3
_B = 16


def _yolo_body(l_ref, tgt_ref, in_ref, pred_ref, yt_ref, noobj_ref, bls_ref,
               meta_i, meta_f):
    b = pl.program_id(0)

    attr_col = jax.lax.broadcasted_iota(jnp.int32, (_ATTRS, _HW), 0)
    for a in range(_NA):
        x = in_ref[0, a]
        sig = 1.0 / (1.0 + jnp.exp(-x))
        sel = jnp.where((attr_col == 2) | (attr_col == 3), x, sig)
        pred_ref[0, a] = sel.T
        yt_ref[0, a] = jnp.zeros((_HW, _ATTRS), jnp.float32)
    noobj_ref[0] = jnp.ones((_NA, _H, _W), jnp.float32)
    bls_ref[0] = jnp.zeros((_NA, _H, _W), jnp.float32)

    aw = [float(_ANCHORS[n, 0] / _STRIDE) for n in range(9)]
    ah = [float(_ANCHORS[n, 1] / _STRIDE) for n in range(9)]

    base = (2 - l_ref[0]) * 3
    for t in range(_NBOX):
        bt0 = tgt_ref[b, t, 0] * _W
        bt1 = tgt_ref[b, t, 1] * _H
        bt2 = tgt_ref[b, t, 2] * _W
        bt3 = tgt_ref[b, t, 3] * _H
        area = bt2 * bt3

        best_iou = jnp.float32(-1.0)
        baw = jnp.float32(aw[0])
        bah = jnp.float32(ah[0])
        best_n = jnp.int32(0)
        for n in range(9):
            inter = jnp.minimum(bt2, aw[n]) * jnp.minimum(bt3, ah[n])
            union = area + aw[n] * ah[n] - inter
            iou = inter / jnp.maximum(union, 1e-12)
            better = iou > best_iou
            best_iou = jnp.where(better, iou, best_iou)
            best_n = jnp.where(better, jnp.int32(n), best_n)
            baw = jnp.where(better, jnp.float32(aw[n]), baw)
            bah = jnp.where(better, jnp.float32(ah[n]), bah)

        i = bt0.astype(jnp.int32)
        j = bt1.astype(jnp.int32)
        fi = i.astype(jnp.float32)
        meta_i[t, 0] = best_n - base
        meta_i[t, 1] = j * _W + i
        meta_i[t, 2] = tgt_ref[b, t, 4].astype(jnp.int32)
        meta_i[t, 3] = i
        meta_i[t, 4] = j
        meta_f[t, 0] = bt0 - fi
        meta_f[t, 1] = bt1 - fi
        meta_f[t, 2] = bt2 / baw
        meta_f[t, 3] = bt3 / bah
        meta_f[t, 4] = area / float(_HW)

    lane_w = jax.lax.broadcasted_iota(jnp.int32, (1, _W), 1)
    attr_row = jax.lax.broadcasted_iota(jnp.int32, (1, _ATTRS), 1)
    for t in range(_NBOX):
        k = meta_i[t, 0]

        @pl.when((k >= 0) & (k < _NA))
        def _(t=t, k=k):
            cell = meta_i[t, 1]
            c = meta_i[t, 2]
            i = meta_i[t, 3]
            j = meta_i[t, 4]
            old = yt_ref[0, k, pl.ds(cell, 1), :]
            ratio = jnp.where(attr_row == 2, meta_f[t, 2],
                              jnp.where(attr_row == 3, meta_f[t, 3], 1.0))
            lr = jnp.log(ratio)
            head = jnp.where(attr_row == 0, meta_f[t, 0],
                             jnp.where(attr_row == 1, meta_f[t, 1],
                                       jnp.where(attr_row == 4, 1.0, lr)))
            new = jnp.where(attr_row < 5, head,
                            jnp.where(attr_row == c + 5, 1.0, old))
            yt_ref[0, k, pl.ds(cell, 1), :] = new
            rown = noobj_ref[0, k, pl.ds(j, 1), :]
            noobj_ref[0, k, pl.ds(j, 1), :] = jnp.where(
                lane_w == i, 0.0, rown)
            rowb = bls_ref[0, k, pl.ds(j, 1), :]
            bls_ref[0, k, pl.ds(j, 1), :] = jnp.where(
                lane_w == i, meta_f[t, 4], rowb)


def _run(l_arr, target, inp2, interpret=False):
    grid_spec = pltpu.PrefetchScalarGridSpec(
        num_scalar_prefetch=2,
        grid=(_B,),
        in_specs=[
            pl.BlockSpec((1, _NA, _ATTRS, _HW), lambda b, *_: (b, 0, 0, 0)),
        ],
        out_specs=[
            pl.BlockSpec((1, _NA, _HW, _ATTRS), lambda b, *_: (b, 0, 0, 0)),
            pl.BlockSpec((1, _NA, _HW, _ATTRS), lambda b, *_: (b, 0, 0, 0)),
            pl.BlockSpec((1, _NA, _H, _W), lambda b, *_: (b, 0, 0, 0)),
            pl.BlockSpec((1, _NA, _H, _W), lambda b, *_: (b, 0, 0, 0)),
        ],
        scratch_shapes=[
            pltpu.SMEM((_NBOX, 5), jnp.int32),
            pltpu.SMEM((_NBOX, 5), jnp.float32),
        ],
    )
    out_shapes = [
        jax.ShapeDtypeStruct((_B, _NA, _HW, _ATTRS), jnp.float32),
        jax.ShapeDtypeStruct((_B, _NA, _HW, _ATTRS), jnp.float32),
        jax.ShapeDtypeStruct((_B, _NA, _H, _W), jnp.float32),
        jax.ShapeDtypeStruct((_B, _NA, _H, _W), jnp.float32),
    ]
    return pl.pallas_call(
        _yolo_body,
        grid_spec=grid_spec,
        out_shape=out_shapes,
        interpret=interpret,
    )(l_arr, target, inp2)


def kernel(l, input, target):
    inp2 = input.reshape(_B, _NA, _ATTRS, _HW)
    l_arr = jnp.asarray(l, jnp.int32).reshape(1)
    predv, ytv, noobj, bls = _run(l_arr, target, inp2)
    pred = predv.reshape(_B, _NA, _H, _W, _ATTRS)
    y_true = ytv.reshape(_B, _NA, _H, _W, _ATTRS)
    return (pred, y_true, noobj, bls)
